# baseline (speedup 1.0000x reference)
import functools

import jax
import jax.numpy as jnp
import numpy as np
from jax import lax
from jax.experimental import pallas as pl
from jax.experimental.pallas import tpu as pltpu
from jax.experimental.pallas import tpu_sc as plsc

N = 10000
E = 320000
D = 128
NREL = 4
NBINS = 3
R = NREL * NBINS
NC = 2
NS = 16
NW = NC * NS
EPW = E // NW
K = 80
NCH = EPW // K
CNT_PAD = 960 * 128
N_PAD = 10240
ROWS_PT = N_PAD // NS


def _idx_body(src_ref, dst_ref, dist_ref, dir_ref, isrc_ref, idst_ref):
    d = dist_ref[...]
    m = jnp.max(d)
    b0 = m * np.float32(np.float32(1.0) / np.float32(3.0))
    b1 = m * np.float32(np.float32(2.0) / np.float32(3.0))
    bins = (b0 < d).astype(jnp.int32) + (b1 < d).astype(jnp.int32)
    et = dir_ref[...] * NBINS + bins
    isrc_ref[...] = et * N + src_ref[...]
    idst_ref[...] = et * N + dst_ref[...]


def _edge_ids(src2, dst2, dist2, dir2):
    return pl.pallas_call(
        _idx_body,
        out_shape=[jax.ShapeDtypeStruct(src2.shape, jnp.int32)] * 2,
    )(src2, dst2, dist2, dir2)


def _cnt_body(idst_hbm, zer_hbm, ones_hbm, cnt_hbm, idx0, idx1, idxs0, idxs1,
              ones_v, stg_v, semi0, semi1, semsc0, semsc1, cnt_sp):
    c = lax.axis_index("c")
    s = lax.axis_index("s")
    wid = c * NS + s
    bufs = ((idx0, idxs0, semi0, semsc0), (idx1, idxs1, semi1, semsc1))
    sl_sp = pl.ds(s * (CNT_PAD // NS), CNT_PAD // NS)
    pltpu.sync_copy(zer_hbm, stg_v)
    pltpu.sync_copy(stg_v, cnt_sp.at[sl_sp])
    pltpu.sync_copy(ones_hbm, ones_v)
    plsc.subcore_barrier()

    def issue_load(i, b):
        idx_v, _, semi, _ = bufs[b]
        pltpu.async_copy(idst_hbm.at[pl.ds(wid * EPW + i * K, K)], idx_v, semi)

    def wait_load(b):
        idx_v, _, semi, _ = bufs[b]
        pltpu.make_async_copy(idst_hbm.at[pl.ds(0, K)], idx_v, semi).wait()

    def scatter(b):
        idx_v, idxs_v, _, semsc = bufs[b]
        for g in range(K // 16):
            idxs_v[pl.ds(g * 16, 16)] = idx_v[pl.ds(g * 16, 16)]
        pltpu.async_copy(ones_v, cnt_sp.at[idxs_v], semsc, add=True)

    def wait_scatter(b):
        _, _, _, semsc = bufs[b]
        pltpu.make_async_copy(ones_v, cnt_sp.at[pl.ds(0, K)], semsc).wait()

    issue_load(0, 0)
    issue_load(1, 1)

    def pair(ii, carry):
        for b in range(2):
            i = 2 * ii + b
            wait_load(b)

            @pl.when(ii > 0)
            def _():
                wait_scatter(b)

            scatter(b)
            if b == 0:
                issue_load(i + 2, 0)
            else:
                @pl.when(ii < (NCH - 1) // 2 - 1)
                def _():
                    issue_load(i + 2, 1)
        return carry

    lax.fori_loop(0, (NCH - 1) // 2, pair, 0)
    wait_load(0)
    wait_scatter(0)
    scatter(0)
    wait_scatter(1)
    wait_scatter(0)
    plsc.subcore_barrier()
    out_sl = pl.ds(c * CNT_PAD + s * (CNT_PAD // NS), CNT_PAD // NS)
    pltpu.sync_copy(cnt_sp.at[sl_sp], stg_v)
    pltpu.sync_copy(stg_v, cnt_hbm.at[out_sl])


def _counts(idst):
    mesh = plsc.VectorSubcoreMesh(
        core_axis_name="c", subcore_axis_name="s", num_cores=NC, num_subcores=NS
    )
    zer = jnp.zeros((CNT_PAD // NS,), jnp.float32)
    ones = jnp.ones((K,), jnp.float32)
    f = pl.kernel(
        _cnt_body,
        out_type=jax.ShapeDtypeStruct((NC * CNT_PAD,), jnp.float32),
        mesh=mesh,
        scratch_types=(
            [pltpu.VMEM((K,), jnp.int32)] * 4
            + [pltpu.VMEM((K,), jnp.float32)]
            + [pltpu.VMEM((CNT_PAD // NS,), jnp.float32)]
            + [pltpu.SemaphoreType.DMA] * 4
            + [pltpu.VMEM_SHARED((CNT_PAD,), jnp.float32)]
        ),
    )
    return f(idst, zer, ones)


def _mm_inv_body(x_ref, w_ref, cnt_ref, o_ref, inv_ref):
    acc = jnp.dot(x_ref[...], w_ref[0], preferred_element_type=jnp.float32)
    o_ref[...] = acc[None]
    inv_ref[...] = 1.0 / jnp.maximum(cnt_ref[0] + cnt_ref[1], 1.0)


def _mm_inv(x, W, cnt3):
    bn = 2000
    bc = cnt3.shape[1] // (N // bn)
    return pl.pallas_call(
        _mm_inv_body,
        grid=(N // bn, R),
        in_specs=[
            pl.BlockSpec((bn, D), lambda n, r: (n, 0)),
            pl.BlockSpec((1, D, D), lambda n, r: (r, 0, 0)),
            pl.BlockSpec((NC, bc, 128), lambda n, r: (0, n, 0)),
        ],
        out_specs=[
            pl.BlockSpec((1, bn, D), lambda n, r: (r, n, 0)),
            pl.BlockSpec((bc, 128), lambda n, r: (n, 0)),
        ],
        out_shape=[
            jax.ShapeDtypeStruct((R, N, D), jnp.float32),
            jax.ShapeDtypeStruct((cnt3.shape[1], 128), jnp.float32),
        ],
    )(x, W, cnt3)


def _agg_body(xw_hbm, isrc_hbm, idst_hbm, dst_hbm, inv_hbm, zer_hbm, acc_hbm,
              isrc0, isrc1, idst0, idst1, dst0, dst1, dsts0, dsts1, w0, w1,
              rows0, rows1, semi0, semi1, semg0, semg1, semw0, semw1,
              semsc0, semsc1, acc_sp):
    c = lax.axis_index("c")
    s = lax.axis_index("s")
    wid = c * NS + s
    bufs = (
        (isrc0, idst0, dst0, dsts0, w0, rows0, semi0, semg0, semw0, semsc0),
        (isrc1, idst1, dst1, dsts1, w1, rows1, semi1, semg1, semw1, semsc1),
    )

    def issue_load(i, b):
        isrc_v, idst_v, dst_v, _, _, _, semi, _, _, _ = bufs[b]
        base = wid * EPW + i * K
        pltpu.async_copy(isrc_hbm.at[pl.ds(base, K)], isrc_v, semi)
        pltpu.async_copy(idst_hbm.at[pl.ds(base, K)], idst_v, semi)
        pltpu.async_copy(dst_hbm.at[pl.ds(base, K)], dst_v, semi)

    def wait_load(b):
        isrc_v, idst_v, dst_v, _, _, _, semi, _, _, _ = bufs[b]
        z = pl.ds(0, K)
        pltpu.make_async_copy(isrc_hbm.at[z], isrc_v, semi).wait()
        pltpu.make_async_copy(idst_hbm.at[z], idst_v, semi).wait()
        pltpu.make_async_copy(dst_hbm.at[z], dst_v, semi).wait()

    def issue_gather(b):
        isrc_v, idst_v, _, _, w_v, rows_v, _, semg, semw, _ = bufs[b]
        pltpu.async_copy(xw_hbm.at[isrc_v], rows_v, semg)
        pltpu.async_copy(inv_hbm.at[idst_v], w_v, semw)

    def wait_gather(b):
        _, _, _, _, w_v, rows_v, _, semg, semw, _ = bufs[b]
        pltpu.make_async_copy(xw_hbm.at[pl.ds(0, K)], rows_v, semg).wait()
        pltpu.make_async_copy(inv_hbm.at[pl.ds(0, K)], w_v, semw).wait()

    def scale(b):
        _, _, dst_v, dsts_v, w_v, rows_v, _, _, _, _ = bufs[b]

        def scale_grp(g, carry2):
            wv = w_v[pl.ds(g * 16, 16)]
            dsts_v[pl.ds(g * 16, 16)] = dst_v[pl.ds(g * 16, 16)]
            for t in range(16):
                k = g * 16 + t
                wk = wv[t]
                for j in range(D // 16):
                    sl = pl.ds(j * 16, 16)
                    rows_v[k, sl] = rows_v[k, sl] * wk
            return carry2

        lax.fori_loop(0, K // 16, scale_grp, 0)

    def issue_scatter(b):
        _, _, _, dsts_v, _, rows_v, _, _, _, semsc = bufs[b]
        pltpu.async_copy(rows_v, acc_sp.at[dsts_v], semsc, add=True)

    def wait_scatter(b):
        _, _, _, _, _, rows_v, _, _, _, semsc = bufs[b]
        pltpu.make_async_copy(rows_v, acc_sp.at[pl.ds(0, K)], semsc).wait()

    pltpu.sync_copy(zer_hbm, rows0)
    for p in range(ROWS_PT // K):
        pltpu.sync_copy(rows0, acc_sp.at[pl.ds(s * ROWS_PT + p * K, K)])
    plsc.subcore_barrier()

    pltpu.sync_copy(isrc_hbm.at[pl.ds(wid * EPW, K)], isrc0)
    pltpu.sync_copy(idst_hbm.at[pl.ds(wid * EPW, K)], idst0)
    pltpu.sync_copy(dst_hbm.at[pl.ds(wid * EPW, K)], dst0)
    issue_load(1, 1)
    issue_gather(0)

    def pair(ii, carry):
        for b in range(2):
            i = 2 * ii + b
            wait_load(1 - b)
            if b == 0:
                @pl.when(ii > 0)
                def _():
                    wait_scatter(1)
            else:
                wait_scatter(0)
            issue_gather(1 - b)
            wait_gather(b)
            scale(b)
            issue_scatter(b)
            if b == 0:
                issue_load(i + 2, 0)
            else:
                @pl.when(ii < (NCH - 1) // 2 - 1)
                def _():
                    issue_load(i + 2, 1)
        return carry

    lax.fori_loop(0, (NCH - 1) // 2, pair, 0)
    wait_gather(0)
    scale(0)
    issue_scatter(0)
    wait_scatter(1)
    wait_scatter(0)

    plsc.subcore_barrier()
    for p in range(ROWS_PT // K):
        row_sl = pl.ds(s * ROWS_PT + p * K, K)
        pltpu.sync_copy(acc_sp.at[row_sl], rows0)
        pltpu.sync_copy(rows0, acc_hbm.at[c, row_sl])


def _aggregate(xw, isrc, idst, dst, inv):
    mesh = plsc.VectorSubcoreMesh(
        core_axis_name="c", subcore_axis_name="s", num_cores=NC, num_subcores=NS
    )
    zer = jnp.zeros((K, D), jnp.float32)
    f = pl.kernel(
        _agg_body,
        out_type=jax.ShapeDtypeStruct((NC, N_PAD, D), jnp.float32),
        mesh=mesh,
        scratch_types=(
            [pltpu.VMEM((K,), jnp.int32)] * 8
            + [pltpu.VMEM((K,), jnp.float32)] * 2
            + [pltpu.VMEM((K, D), jnp.float32)] * 2
            + [pltpu.SemaphoreType.DMA] * 8
            + [pltpu.VMEM_SHARED((N_PAD, D), jnp.float32)]
        ),
    )
    return f(xw, isrc, idst, dst, inv, zer)


def _comb_mm_body(acc_ref, x_ref, root_ref, b_ref, w2_ref, h_ref, xw_ref):
    h = (acc_ref[0] + acc_ref[1] + b_ref[...]
         + jnp.dot(x_ref[...], root_ref[...],
                   preferred_element_type=jnp.float32))
    h = jnp.maximum(h, 0.0)
    h_ref[...] = h
    for r in range(R):
        xw_ref[r] = jnp.dot(h, w2_ref[r], preferred_element_type=jnp.float32)


def _comb_mm(acc, x, root, b, W2):
    bn = 2000
    return pl.pallas_call(
        _comb_mm_body,
        grid=(N // bn,),
        in_specs=[
            pl.BlockSpec((NC, bn, D), lambda n: (0, n, 0)),
            pl.BlockSpec((bn, D), lambda n: (n, 0)),
            pl.BlockSpec((D, D), lambda n: (0, 0)),
            pl.BlockSpec((1, D), lambda n: (0, 0)),
            pl.BlockSpec((R, D, D), lambda n: (0, 0, 0)),
        ],
        out_specs=[
            pl.BlockSpec((bn, D), lambda n: (n, 0)),
            pl.BlockSpec((R, bn, D), lambda n: (0, n, 0)),
        ],
        out_shape=[
            jax.ShapeDtypeStruct((N, D), jnp.float32),
            jax.ShapeDtypeStruct((R, N, D), jnp.float32),
        ],
    )(acc, x, root, b, W2)


def _comb_body(acc_ref, x_ref, root_ref, b_ref, o_ref, *, relu):
    v = (acc_ref[0] + acc_ref[1] + b_ref[...]
         + jnp.dot(x_ref[...], root_ref[...],
                   preferred_element_type=jnp.float32))
    o_ref[...] = jnp.maximum(v, 0.0) if relu else v


def _combine(acc, x, root, b, relu):
    bn = 2000
    return pl.pallas_call(
        functools.partial(_comb_body, relu=relu),
        grid=(N // bn,),
        in_specs=[
            pl.BlockSpec((NC, bn, D), lambda n: (0, n, 0)),
            pl.BlockSpec((bn, D), lambda n: (n, 0)),
            pl.BlockSpec((D, D), lambda n: (0, 0)),
            pl.BlockSpec((1, D), lambda n: (0, 0)),
        ],
        out_specs=pl.BlockSpec((bn, D), lambda n: (n, 0)),
        out_shape=jax.ShapeDtypeStruct((N, D), jnp.float32),
    )(acc, x, root, b)


def kernel(x, edge_index, edge_attr, W1, root1, b1, W2, root2, b2, bias):
    src2 = edge_index[0].reshape(E // 128, 128)
    dst2 = edge_index[1].reshape(E // 128, 128)
    dist2 = edge_attr[:, 0].reshape(E // 128, 128)
    dir2 = edge_attr[:, 1].astype(jnp.int32).reshape(E // 128, 128)

    isrc2, idst2 = _edge_ids(src2, dst2, dist2, dir2)
    isrc = isrc2.reshape(E)
    idst = idst2.reshape(E)
    dst = edge_index[1]

    cnt = _counts(idst)
    xw1, inv2d = _mm_inv(x, W1, cnt.reshape(NC, CNT_PAD // 128, 128))
    inv = inv2d.reshape(CNT_PAD)
    acc1 = _aggregate(xw1.reshape(R * N, D), isrc, idst, dst, inv)
    h, xw2 = _comb_mm(acc1, x, root1, b1.reshape(1, D), W2)

    acc2 = _aggregate(xw2.reshape(R * N, D), isrc, idst, dst, inv)
    out = _combine(acc2, h, root2, (b2 + bias).reshape(1, D), relu=False)
    return out


# speedup vs baseline: 1.0310x; 1.0310x over previous
import functools

import jax
import jax.numpy as jnp
import numpy as np
from jax import lax
from jax.experimental import pallas as pl
from jax.experimental.pallas import tpu as pltpu
from jax.experimental.pallas import tpu_sc as plsc

N = 10000
E = 320000
D = 128
NREL = 4
NBINS = 3
R = NREL * NBINS
NC = 2
NS = 16
NW = NC * NS
EPW = E // NW
K = 80
NCH = EPW // K
CNT_PAD = 940 * 128
N_PAD = 10240
ROWS_PT = N_PAD // NS


def _idx_body(src_ref, dst_ref, dist_ref, dir_ref, isrc_ref, idst_ref):
    d = dist_ref[...]
    m = jnp.max(d)
    b0 = m * np.float32(np.float32(1.0) / np.float32(3.0))
    b1 = m * np.float32(np.float32(2.0) / np.float32(3.0))
    bins = (b0 < d).astype(jnp.int32) + (b1 < d).astype(jnp.int32)
    et = dir_ref[...] * NBINS + bins
    isrc_ref[...] = et * N + src_ref[...]
    idst_ref[...] = et * N + dst_ref[...]


def _edge_ids(src2, dst2, dist2, dir2):
    return pl.pallas_call(
        _idx_body,
        out_shape=[jax.ShapeDtypeStruct(src2.shape, jnp.int32)] * 2,
    )(src2, dst2, dist2, dir2)


def _cnt_body(idst_hbm, zer_hbm, ones_hbm, cnt_hbm, idx0, idx1, idxs0, idxs1,
              ones_v, stg_v, semi0, semi1, semsc0, semsc1, cnt_sp):
    c = lax.axis_index("c")
    s = lax.axis_index("s")
    wid = c * NS + s
    bufs = ((idx0, idxs0, semi0, semsc0), (idx1, idxs1, semi1, semsc1))
    sl_sp = pl.ds(s * (CNT_PAD // NS), CNT_PAD // NS)
    pltpu.sync_copy(zer_hbm, stg_v)
    pltpu.sync_copy(stg_v, cnt_sp.at[sl_sp])
    pltpu.sync_copy(ones_hbm, ones_v)
    plsc.subcore_barrier()

    def issue_load(i, b):
        idx_v, _, semi, _ = bufs[b]
        pltpu.async_copy(idst_hbm.at[pl.ds(wid * EPW + i * K, K)], idx_v, semi)

    def wait_load(b):
        idx_v, _, semi, _ = bufs[b]
        pltpu.make_async_copy(idst_hbm.at[pl.ds(0, K)], idx_v, semi).wait()

    def scatter(b):
        idx_v, idxs_v, _, semsc = bufs[b]
        for g in range(K // 16):
            idxs_v[pl.ds(g * 16, 16)] = idx_v[pl.ds(g * 16, 16)]
        pltpu.async_copy(ones_v, cnt_sp.at[idxs_v], semsc, add=True)

    def wait_scatter(b):
        _, _, _, semsc = bufs[b]
        pltpu.make_async_copy(ones_v, cnt_sp.at[pl.ds(0, K)], semsc).wait()

    issue_load(0, 0)
    issue_load(1, 1)

    def pair(ii, carry):
        for b in range(2):
            i = 2 * ii + b
            wait_load(b)

            @pl.when(ii > 0)
            def _():
                wait_scatter(b)

            scatter(b)
            if b == 0:
                issue_load(i + 2, 0)
            else:
                @pl.when(ii < (NCH - 1) // 2 - 1)
                def _():
                    issue_load(i + 2, 1)
        return carry

    lax.fori_loop(0, (NCH - 1) // 2, pair, 0)
    wait_load(0)
    wait_scatter(0)
    scatter(0)
    wait_scatter(1)
    wait_scatter(0)
    plsc.subcore_barrier()
    out_sl = pl.ds(c * CNT_PAD + s * (CNT_PAD // NS), CNT_PAD // NS)
    pltpu.sync_copy(cnt_sp.at[sl_sp], stg_v)
    pltpu.sync_copy(stg_v, cnt_hbm.at[out_sl])


def _counts(idst):
    mesh = plsc.VectorSubcoreMesh(
        core_axis_name="c", subcore_axis_name="s", num_cores=NC, num_subcores=NS
    )
    zer = jnp.zeros((CNT_PAD // NS,), jnp.float32)
    ones = jnp.ones((K,), jnp.float32)
    f = pl.kernel(
        _cnt_body,
        out_type=jax.ShapeDtypeStruct((NC * CNT_PAD,), jnp.float32),
        mesh=mesh,
        scratch_types=(
            [pltpu.VMEM((K,), jnp.int32)] * 4
            + [pltpu.VMEM((K,), jnp.float32)]
            + [pltpu.VMEM((CNT_PAD // NS,), jnp.float32)]
            + [pltpu.SemaphoreType.DMA] * 4
            + [pltpu.VMEM_SHARED((CNT_PAD,), jnp.float32)]
        ),
    )
    return f(idst, zer, ones)


def _inv_body(cnt_ref, inv_ref):
    inv_ref[...] = 1.0 / jnp.maximum(cnt_ref[0] + cnt_ref[1], 1.0)


def _inv_counts(cnt3):
    return pl.pallas_call(
        _inv_body,
        out_shape=jax.ShapeDtypeStruct((cnt3.shape[1], 128), jnp.float32),
    )(cnt3)


def _mm_body(x_ref, w_ref, o_ref):
    acc = jnp.dot(x_ref[...], w_ref[0], preferred_element_type=jnp.float32)
    o_ref[...] = acc[None]


def _rel_matmul(x, W):
    bn = 2000
    return pl.pallas_call(
        _mm_body,
        grid=(R, N // bn),
        in_specs=[
            pl.BlockSpec((bn, D), lambda r, n: (n, 0)),
            pl.BlockSpec((1, D, D), lambda r, n: (r, 0, 0)),
        ],
        out_specs=pl.BlockSpec((1, bn, D), lambda r, n: (r, n, 0)),
        out_shape=jax.ShapeDtypeStruct((R, N, D), jnp.float32),
    )(x, W)


def _agg_body(xw_hbm, isrc_hbm, idst_hbm, dst_hbm, inv_hbm, zer_hbm, acc_hbm,
              isrc0, isrc1, idst0, idst1, dst0, dst1, dsts0, dsts1, w0, w1,
              rows0, rows1, semi0, semi1, semg0, semg1, semw0, semw1,
              semsc0, semsc1, acc_sp):
    c = lax.axis_index("c")
    s = lax.axis_index("s")
    wid = c * NS + s
    bufs = (
        (isrc0, idst0, dst0, dsts0, w0, rows0, semi0, semg0, semw0, semsc0),
        (isrc1, idst1, dst1, dsts1, w1, rows1, semi1, semg1, semw1, semsc1),
    )

    def issue_load(i, b):
        isrc_v, idst_v, dst_v, _, _, _, semi, _, _, _ = bufs[b]
        base = wid * EPW + i * K
        pltpu.async_copy(isrc_hbm.at[pl.ds(base, K)], isrc_v, semi)
        pltpu.async_copy(idst_hbm.at[pl.ds(base, K)], idst_v, semi)
        pltpu.async_copy(dst_hbm.at[pl.ds(base, K)], dst_v, semi)

    def wait_load(b):
        isrc_v, idst_v, dst_v, _, _, _, semi, _, _, _ = bufs[b]
        z = pl.ds(0, K)
        pltpu.make_async_copy(isrc_hbm.at[z], isrc_v, semi).wait()
        pltpu.make_async_copy(idst_hbm.at[z], idst_v, semi).wait()
        pltpu.make_async_copy(dst_hbm.at[z], dst_v, semi).wait()

    def issue_gather(b):
        isrc_v, idst_v, _, _, w_v, rows_v, _, semg, semw, _ = bufs[b]
        pltpu.async_copy(xw_hbm.at[isrc_v], rows_v, semg)
        pltpu.async_copy(inv_hbm.at[idst_v], w_v, semw)

    def wait_gather(b):
        _, _, _, _, w_v, rows_v, _, semg, semw, _ = bufs[b]
        pltpu.make_async_copy(xw_hbm.at[pl.ds(0, K)], rows_v, semg).wait()
        pltpu.make_async_copy(inv_hbm.at[pl.ds(0, K)], w_v, semw).wait()

    def scale(b):
        _, _, dst_v, dsts_v, w_v, rows_v, _, _, _, _ = bufs[b]

        def scale_grp(g, carry2):
            wv = w_v[pl.ds(g * 16, 16)]
            dsts_v[pl.ds(g * 16, 16)] = dst_v[pl.ds(g * 16, 16)]
            for t in range(16):
                k = g * 16 + t
                wk = wv[t]
                for j in range(D // 16):
                    sl = pl.ds(j * 16, 16)
                    rows_v[k, sl] = rows_v[k, sl] * wk
            return carry2

        lax.fori_loop(0, K // 16, scale_grp, 0)

    def issue_scatter(b):
        _, _, _, dsts_v, _, rows_v, _, _, _, semsc = bufs[b]
        pltpu.async_copy(rows_v, acc_sp.at[dsts_v], semsc, add=True)

    def wait_scatter(b):
        _, _, _, _, _, rows_v, _, _, _, semsc = bufs[b]
        pltpu.make_async_copy(rows_v, acc_sp.at[pl.ds(0, K)], semsc).wait()

    pltpu.sync_copy(zer_hbm, rows0)
    for p in range(ROWS_PT // K):
        pltpu.sync_copy(rows0, acc_sp.at[pl.ds(s * ROWS_PT + p * K, K)])
    plsc.subcore_barrier()

    pltpu.sync_copy(isrc_hbm.at[pl.ds(wid * EPW, K)], isrc0)
    pltpu.sync_copy(idst_hbm.at[pl.ds(wid * EPW, K)], idst0)
    pltpu.sync_copy(dst_hbm.at[pl.ds(wid * EPW, K)], dst0)
    issue_load(1, 1)
    issue_gather(0)

    def pair(ii, carry):
        for b in range(2):
            i = 2 * ii + b
            wait_load(1 - b)
            if b == 0:
                @pl.when(ii > 0)
                def _():
                    wait_scatter(1)
            else:
                wait_scatter(0)
            issue_gather(1 - b)
            wait_gather(b)
            scale(b)
            issue_scatter(b)
            if b == 0:
                issue_load(i + 2, 0)
            else:
                @pl.when(ii < (NCH - 1) // 2 - 1)
                def _():
                    issue_load(i + 2, 1)
        return carry

    lax.fori_loop(0, (NCH - 1) // 2, pair, 0)
    wait_gather(0)
    scale(0)
    issue_scatter(0)
    wait_scatter(1)
    wait_scatter(0)

    plsc.subcore_barrier()
    for p in range(ROWS_PT // K):
        row_sl = pl.ds(s * ROWS_PT + p * K, K)
        pltpu.sync_copy(acc_sp.at[row_sl], rows0)
        pltpu.sync_copy(rows0, acc_hbm.at[c, row_sl])


def _aggregate(xw, isrc, idst, dst, inv):
    mesh = plsc.VectorSubcoreMesh(
        core_axis_name="c", subcore_axis_name="s", num_cores=NC, num_subcores=NS
    )
    zer = jnp.zeros((K, D), jnp.float32)
    f = pl.kernel(
        _agg_body,
        out_type=jax.ShapeDtypeStruct((NC, N_PAD, D), jnp.float32),
        mesh=mesh,
        scratch_types=(
            [pltpu.VMEM((K,), jnp.int32)] * 8
            + [pltpu.VMEM((K,), jnp.float32)] * 2
            + [pltpu.VMEM((K, D), jnp.float32)] * 2
            + [pltpu.SemaphoreType.DMA] * 8
            + [pltpu.VMEM_SHARED((N_PAD, D), jnp.float32)]
        ),
    )
    return f(xw, isrc, idst, dst, inv, zer)


def _comb_mm_body(acc_ref, x_ref, root_ref, b_ref, w2_ref, h_ref, xw_ref):
    h = (acc_ref[0] + acc_ref[1] + b_ref[...]
         + jnp.dot(x_ref[...], root_ref[...],
                   preferred_element_type=jnp.float32))
    h = jnp.maximum(h, 0.0)
    h_ref[...] = h
    for r in range(R):
        xw_ref[r] = jnp.dot(h, w2_ref[r], preferred_element_type=jnp.float32)


def _comb_mm(acc, x, root, b, W2):
    bn = 2000
    return pl.pallas_call(
        _comb_mm_body,
        grid=(N // bn,),
        in_specs=[
            pl.BlockSpec((NC, bn, D), lambda n: (0, n, 0)),
            pl.BlockSpec((bn, D), lambda n: (n, 0)),
            pl.BlockSpec((D, D), lambda n: (0, 0)),
            pl.BlockSpec((1, D), lambda n: (0, 0)),
            pl.BlockSpec((R, D, D), lambda n: (0, 0, 0)),
        ],
        out_specs=[
            pl.BlockSpec((bn, D), lambda n: (n, 0)),
            pl.BlockSpec((R, bn, D), lambda n: (0, n, 0)),
        ],
        out_shape=[
            jax.ShapeDtypeStruct((N, D), jnp.float32),
            jax.ShapeDtypeStruct((R, N, D), jnp.float32),
        ],
    )(acc, x, root, b, W2)


def _comb_body(acc_ref, x_ref, root_ref, b_ref, o_ref, *, relu):
    v = (acc_ref[0] + acc_ref[1] + b_ref[...]
         + jnp.dot(x_ref[...], root_ref[...],
                   preferred_element_type=jnp.float32))
    o_ref[...] = jnp.maximum(v, 0.0) if relu else v


def _combine(acc, x, root, b, relu):
    bn = 2000
    return pl.pallas_call(
        functools.partial(_comb_body, relu=relu),
        grid=(N // bn,),
        in_specs=[
            pl.BlockSpec((NC, bn, D), lambda n: (0, n, 0)),
            pl.BlockSpec((bn, D), lambda n: (n, 0)),
            pl.BlockSpec((D, D), lambda n: (0, 0)),
            pl.BlockSpec((1, D), lambda n: (0, 0)),
        ],
        out_specs=pl.BlockSpec((bn, D), lambda n: (n, 0)),
        out_shape=jax.ShapeDtypeStruct((N, D), jnp.float32),
    )(acc, x, root, b)


def kernel(x, edge_index, edge_attr, W1, root1, b1, W2, root2, b2, bias):
    src2 = edge_index[0].reshape(E // 128, 128)
    dst2 = edge_index[1].reshape(E // 128, 128)
    dist2 = edge_attr[:, 0].reshape(E // 128, 128)
    dir2 = edge_attr[:, 1].astype(jnp.int32).reshape(E // 128, 128)

    isrc2, idst2 = _edge_ids(src2, dst2, dist2, dir2)
    isrc = isrc2.reshape(E)
    idst = idst2.reshape(E)
    dst = edge_index[1]

    cnt = _counts(idst)
    inv = _inv_counts(cnt.reshape(NC, CNT_PAD // 128, 128)).reshape(CNT_PAD)

    xw1 = _rel_matmul(x, W1).reshape(R * N, D)
    acc1 = _aggregate(xw1, isrc, idst, dst, inv)
    h, xw2 = _comb_mm(acc1, x, root1, b1.reshape(1, D), W2)

    acc2 = _aggregate(xw2.reshape(R * N, D), isrc, idst, dst, inv)
    out = _combine(acc2, h, root2, (b2 + bias).reshape(1, D), relu=False)
    return out
